# Initial kernel scaffold; baseline (speedup 1.0000x reference)
import jax
import jax.numpy as jnp
from jax.experimental import pallas as pl


def kernel(entity_emb, user_emb, edge_index, edge_type, interact_mat, weight):
    raise NotImplementedError("write your pallas kernel here")


# speedup vs baseline: 4.4968x; 4.4968x over previous
import functools
import jax
import jax.numpy as jnp
from jax import lax
from jax.experimental import pallas as pl
from jax.experimental.pallas import tpu as pltpu
from jax.experimental.pallas import tpu_sc as plsc

NC, NS, L = 2, 16, 16
NW = NC * NS
B = 80
EPS = 1e-16
HIGH = lax.Precision.HIGHEST


def _dedup_combine(hsc, vsc, hv, vv, lanes, is_add):
  hvf = hv.astype(jnp.float32)
  hsc[pl.ds(0, L)] = hvf
  vsc[pl.ds(0, L)] = vv
  acc = vv
  first = jnp.ones((L,), jnp.bool_)
  for s in range(1, L):
    idx = (lanes + s) & (L - 1)
    hr = plsc.load_gather(hsc, [idx])
    vr = plsc.load_gather(vsc, [idx])
    same = hr == hvf
    comb = acc + vr if is_add else jnp.maximum(acc, vr)
    acc = jnp.where(same, comb, acc)
    first = first & jnp.logical_not(same & (idx < lanes))
  return acc, first


def _scatter_max(tab, hsc, vsc, hv, vv, lanes):
  acc, first = _dedup_combine(hsc, vsc, hv, vv, lanes, False)
  cur = plsc.load_gather(tab, [hv])
  plsc.store_scatter(tab, [hv], jnp.maximum(acc, cur), mask=first)


def _scatter_add(tab, hsc, vsc, hv, vv, lanes):
  acc, first = _dedup_combine(hsc, vsc, hv, vv, lanes, True)
  cur = plsc.load_gather(tab, [hv])
  plsc.store_scatter(tab, [hv], cur + acc, mask=first)


def _q_body(emb_ref, w_ref, q_ref):
  e2 = emb_ref[...] * emb_ref[...]
  w2 = w_ref[...] * w_ref[...]
  q_ref[...] = lax.dot_general(e2, w2, (((1,), (1,)), ((), ())),
                               precision=HIGH,
                               preferred_element_type=jnp.float32)


def _q_table(entity_emb, weight):
  N, _ = entity_emb.shape
  R = weight.shape[0]
  return pl.pallas_call(
      _q_body,
      out_shape=jax.ShapeDtypeStruct((N, R), jnp.float32),
  )(entity_emb, weight)


def _maxred_body(parts_ref, out_ref):
  out_ref[...] = jnp.max(parts_ref[...], axis=0)


def _reduce_max(parts):
  _, N = parts.shape
  return pl.pallas_call(
      _maxred_body,
      out_shape=jax.ShapeDtypeStruct((N,), jnp.float32),
  )(parts)


def _make_phase_a(N, R, E):
  EP = E // NW
  CH = EP // B

  def body(q_hbm, head_hbm, hq_hbm, tq_hbm, w_hbm, smax_hbm,
           headv, hqv, tqv, qh, qt, wbuf, smax_loc, hsc, vsc, sem):
    cid = lax.axis_index("c")
    sid = lax.axis_index("s")
    wid = cid * NS + sid
    base = wid * EP
    lanes = lax.iota(jnp.int32, L)
    zeros = jnp.zeros((L,), jnp.float32)

    def zinit(i, c):
      smax_loc[pl.ds(i * L, L)] = zeros
      return c
    lax.fori_loop(0, N // L, zinit, 0)

    def chunk(k, c):
      off = base + k * B
      pltpu.sync_copy(head_hbm.at[pl.ds(off, B)], headv)
      pltpu.sync_copy(hq_hbm.at[pl.ds(off, B)], hqv)
      pltpu.sync_copy(tq_hbm.at[pl.ds(off, B)], tqv)
      pltpu.async_copy(q_hbm.at[hqv], qh, sem).wait()
      pltpu.async_copy(q_hbm.at[tqv], qt, sem).wait()
      for j in range(B // L):
        qhv = qh[pl.ds(j * L, L)]
        qtv = qt[pl.ds(j * L, L)]
        wv = qhv * qtv
        wbuf[pl.ds(j * L, L)] = wv
        hv = headv[pl.ds(j * L, L)]
        _scatter_max(smax_loc, hsc, vsc, hv, wv, lanes)
      pltpu.sync_copy(wbuf, w_hbm.at[pl.ds(off, B)])
      return c
    lax.fori_loop(0, CH, chunk, 0)
    pltpu.sync_copy(smax_loc, smax_hbm.at[wid])

  mesh = plsc.VectorSubcoreMesh(core_axis_name="c", subcore_axis_name="s",
                                num_cores=NC, num_subcores=NS)
  return pl.kernel(
      body,
      out_type=[jax.ShapeDtypeStruct((E,), jnp.float32),
                jax.ShapeDtypeStruct((NW, N), jnp.float32)],
      mesh=mesh,
      compiler_params=pltpu.CompilerParams(needs_layout_passes=False),
      scratch_types=[
          pltpu.VMEM((B,), jnp.int32),
          pltpu.VMEM((B,), jnp.int32),
          pltpu.VMEM((B,), jnp.int32),
          pltpu.VMEM((B,), jnp.float32),
          pltpu.VMEM((B,), jnp.float32),
          pltpu.VMEM((B,), jnp.float32),
          pltpu.VMEM((N,), jnp.float32),
          pltpu.VMEM((128,), jnp.float32),
          pltpu.VMEM((128,), jnp.float32),
          pltpu.SemaphoreType.DMA,
      ],
  )


def _make_phase_c(N, D, R, E):
  EP = E // NW
  CH = EP // B
  STRIPE = (N // NS) // 8 * 8
  TAIL = N - NS * STRIPE

  def body(emb_hbm, wt_hbm, head_hbm, tail_hbm, rel_hbm, we_hbm, smax_hbm,
           zrows_hbm, agg_hbm, sump_hbm,
           headv, tailv, relv, wv, pbuf, rows, orows, smax_loc, ssum_loc,
           hsc, vsc, wtab, agg_sh, sem):
    cid = lax.axis_index("c")
    sid = lax.axis_index("s")
    wid = cid * NS + sid
    base = wid * EP
    lanes = lax.iota(jnp.int32, L)
    zeros = jnp.zeros((L,), jnp.float32)

    pltpu.sync_copy(smax_hbm, smax_loc)
    pltpu.sync_copy(wt_hbm, wtab)

    def zinit(i, c):
      ssum_loc[pl.ds(i * L, L)] = zeros
      return c
    lax.fori_loop(0, N // L, zinit, 0)

    pltpu.sync_copy(zrows_hbm, agg_sh.at[pl.ds(sid * STRIPE, STRIPE)])

    @pl.when(sid == 0)
    def _():
      pltpu.sync_copy(zrows_hbm.at[pl.ds(0, TAIL)],
                      agg_sh.at[pl.ds(NS * STRIPE, TAIL)])
    plsc.subcore_barrier()

    def chunk(k, c):
      off = base + k * B
      pltpu.sync_copy(head_hbm.at[pl.ds(off, B)], headv)
      pltpu.sync_copy(tail_hbm.at[pl.ds(off, B)], tailv)
      pltpu.sync_copy(rel_hbm.at[pl.ds(off, B)], relv)
      pltpu.sync_copy(we_hbm.at[pl.ds(off, B)], wv)
      pltpu.async_copy(emb_hbm.at[tailv], rows, sem).wait()
      for j in range(B // L):
        hv = headv[pl.ds(j * L, L)]
        wvj = wv[pl.ds(j * L, L)]
        m = plsc.load_gather(smax_loc, [hv])
        p = jnp.exp(wvj - m)
        pbuf[pl.ds(j * L, L)] = p
        _scatter_add(ssum_loc, hsc, vsc, hv, p, lanes)

      def edge(i, c2):
        isp = jnp.zeros((L,), jnp.int32) + i
        psp = plsc.load_gather(pbuf, [isp])
        rsp = plsc.load_gather(relv, [isp]).astype(jnp.int32)
        for jj in range(D // L):
          seg = pl.ds(jj * L, L)
          rowv = rows[i, seg]
          wrow = plsc.load_gather(wtab, [rsp * D + jj * L + lanes])
          orows[i, seg] = psp * rowv * wrow
        return c2
      lax.fori_loop(0, B, edge, 0)
      pltpu.sync_copy(orows, agg_sh.at[headv], add=True)
      return c
    lax.fori_loop(0, CH, chunk, 0)

    plsc.subcore_barrier()
    pltpu.sync_copy(agg_sh.at[pl.ds(sid * STRIPE, STRIPE)],
                    agg_hbm.at[pl.ds(cid * N + sid * STRIPE, STRIPE)])

    @pl.when(sid == 0)
    def _():
      pltpu.sync_copy(agg_sh.at[pl.ds(NS * STRIPE, TAIL)],
                      agg_hbm.at[pl.ds(cid * N + NS * STRIPE, TAIL)])
    pltpu.sync_copy(ssum_loc, sump_hbm.at[wid])

  mesh = plsc.VectorSubcoreMesh(core_axis_name="c", subcore_axis_name="s",
                                num_cores=NC, num_subcores=NS)
  return pl.kernel(
      body,
      out_type=[jax.ShapeDtypeStruct((NC * N, D), jnp.float32),
                jax.ShapeDtypeStruct((NW, N), jnp.float32)],
      mesh=mesh,
      compiler_params=pltpu.CompilerParams(needs_layout_passes=False),
      scratch_types=[
          pltpu.VMEM((B,), jnp.int32),
          pltpu.VMEM((B,), jnp.int32),
          pltpu.VMEM((B,), jnp.float32),
          pltpu.VMEM((B,), jnp.float32),
          pltpu.VMEM((B,), jnp.float32),
          pltpu.VMEM((B, D), jnp.float32),
          pltpu.VMEM((B, D), jnp.float32),
          pltpu.VMEM((N,), jnp.float32),
          pltpu.VMEM((N,), jnp.float32),
          pltpu.VMEM((128,), jnp.float32),
          pltpu.VMEM((128,), jnp.float32),
          pltpu.VMEM((R * D,), jnp.float32),
          pltpu.VMEM_SHARED((N, D), jnp.float32),
          pltpu.SemaphoreType.DMA,
      ],
  )


def _make_matmuls(U, N, D):
  BU = 256
  UB = U // BU

  def body(im_ref, emb_ref, uemb_ref, uout_ref, dout_ref):
    u = pl.program_id(0)
    im = im_ref[...]
    uout_ref[...] = lax.dot_general(im, emb_ref[...], (((1,), (0,)), ((), ())),
                                    precision=HIGH,
                                    preferred_element_type=jnp.float32)
    prod_d = lax.dot_general(im, uemb_ref[...], (((0,), (0,)), ((), ())),
                             precision=HIGH,
                             preferred_element_type=jnp.float32)

    @pl.when(u == 0)
    def _():
      dout_ref[...] = prod_d

    @pl.when(u != 0)
    def _():
      dout_ref[...] += prod_d

  return pl.pallas_call(
      body,
      grid=(UB,),
      in_specs=[
          pl.BlockSpec((BU, N), lambda u: (u, 0)),
          pl.BlockSpec((N, D), lambda u: (0, 0)),
          pl.BlockSpec((BU, D), lambda u: (u, 0)),
      ],
      out_specs=[
          pl.BlockSpec((BU, D), lambda u: (u, 0)),
          pl.BlockSpec((N, D), lambda u: (0, 0)),
      ],
      out_shape=[jax.ShapeDtypeStruct((U, D), jnp.float32),
                 jax.ShapeDtypeStruct((N, D), jnp.float32)],
  )


def _make_combine(N, D):
  def body(dense_ref, agg_ref, sum_ref, out_ref):
    s = jnp.sum(sum_ref[...], axis=0)
    a = agg_ref[0] + agg_ref[1]
    out_ref[...] = dense_ref[...] + a * (1.0 / (s + EPS))[:, None]

  return pl.pallas_call(
      body,
      out_shape=jax.ShapeDtypeStruct((N, D), jnp.float32),
  )


@jax.jit
def kernel(entity_emb, user_emb, edge_index, edge_type, interact_mat, weight):
  N, D = entity_emb.shape
  U = user_emb.shape[0]
  E = edge_index.shape[1]
  R = weight.shape[0]

  head = edge_index[0]
  tail = edge_index[1]
  rel = edge_type - 1

  q = _q_table(entity_emb, weight)
  hq_idx = head * R + rel
  tq_idx = tail * R + rel
  w_e, smax_parts = _make_phase_a(N, R, E)(q.reshape(-1), head, hq_idx, tq_idx)
  seg_max = _reduce_max(smax_parts)
  zrows = jnp.zeros(((N // NS) // 8 * 8, D), jnp.float32)
  agg_flat, sum_parts = _make_phase_c(N, D, R, E)(
      entity_emb, weight.reshape(-1), head, tail, rel.astype(jnp.float32),
      w_e, seg_max, zrows)
  agg_parts = agg_flat.reshape(NC, N, D)
  user_agg, dense = _make_matmuls(U, N, D)(interact_mat, entity_emb, user_emb)
  entity_agg = _make_combine(N, D)(dense, agg_parts, sum_parts)
  return entity_agg, user_agg


# speedup vs baseline: 7.9944x; 1.7778x over previous
import functools
import jax
import jax.numpy as jnp
from jax import lax
from jax.experimental import pallas as pl
from jax.experimental.pallas import tpu as pltpu
from jax.experimental.pallas import tpu_sc as plsc

NC, NS, L = 2, 16, 16
NW = NC * NS
B = 80
EPS = 1e-16
HIGH = lax.Precision.HIGHEST


def _dedup_combine(hsc, vsc, hv, vv, lanes, is_add):
  hvf = hv.astype(jnp.float32)
  hsc[pl.ds(0, L)] = hvf
  vsc[pl.ds(0, L)] = vv
  acc = vv
  first = jnp.ones((L,), jnp.bool_)
  for s in range(1, L):
    idx = (lanes + s) & (L - 1)
    hr = plsc.load_gather(hsc, [idx])
    vr = plsc.load_gather(vsc, [idx])
    same = hr == hvf
    comb = acc + vr if is_add else jnp.maximum(acc, vr)
    acc = jnp.where(same, comb, acc)
    first = first & jnp.logical_not(same & (idx < lanes))
  return acc, first


def _scatter_max(tab, hsc, vsc, hv, vv, lanes):
  acc, first = _dedup_combine(hsc, vsc, hv, vv, lanes, False)
  cur = plsc.load_gather(tab, [hv])
  plsc.store_scatter(tab, [hv], jnp.maximum(acc, cur), mask=first)


def _scatter_add(tab, hsc, vsc, hv, vv, lanes):
  acc, first = _dedup_combine(hsc, vsc, hv, vv, lanes, True)
  cur = plsc.load_gather(tab, [hv])
  plsc.store_scatter(tab, [hv], cur + acc, mask=first)


def _q_body(emb_ref, w_ref, q_ref):
  e2 = emb_ref[...] * emb_ref[...]
  w2 = w_ref[...] * w_ref[...]
  q_ref[...] = lax.dot_general(e2, w2, (((1,), (1,)), ((), ())),
                               precision=HIGH,
                               preferred_element_type=jnp.float32)


def _q_table(entity_emb, weight):
  N, _ = entity_emb.shape
  R = weight.shape[0]
  return pl.pallas_call(
      _q_body,
      out_shape=jax.ShapeDtypeStruct((N, R), jnp.float32),
  )(entity_emb, weight)


def _maxred_body(parts_ref, out_ref):
  out_ref[...] = jnp.max(parts_ref[...], axis=0)


def _reduce_max(parts):
  _, N = parts.shape
  return pl.pallas_call(
      _maxred_body,
      out_shape=jax.ShapeDtypeStruct((N,), jnp.float32),
  )(parts)


def _make_phase_a(N, R, E):
  EP = E // NW
  CH = EP // B

  def body(q_hbm, head_hbm, hq_hbm, tq_hbm, w_hbm, smax_hbm,
           headv, hqv, tqv, qh, qt, wbuf, smax_loc, hsc, vsc,
           lsem, qsem, wsem):
    cid = lax.axis_index("c")
    sid = lax.axis_index("s")
    wid = cid * NS + sid
    base = wid * EP
    lanes = lax.iota(jnp.int32, L)
    zeros = jnp.zeros((L,), jnp.float32)

    def zinit(i, c):
      smax_loc[pl.ds(i * L, L)] = zeros
      return c
    lax.fori_loop(0, N // L, zinit, 0)

    def issue_linear(k, b):
      off = base + k * B
      pltpu.async_copy(head_hbm.at[pl.ds(off, B)], headv[b], lsem[b])
      pltpu.async_copy(hq_hbm.at[pl.ds(off, B)], hqv[b], lsem[b])
      pltpu.async_copy(tq_hbm.at[pl.ds(off, B)], tqv[b], lsem[b])

    def wait_linear(k, b):
      off = base + k * B
      pltpu.make_async_copy(head_hbm.at[pl.ds(off, B)], headv[b], lsem[b]).wait()
      pltpu.make_async_copy(hq_hbm.at[pl.ds(off, B)], hqv[b], lsem[b]).wait()
      pltpu.make_async_copy(tq_hbm.at[pl.ds(off, B)], tqv[b], lsem[b]).wait()

    def issue_q(b):
      pltpu.async_copy(q_hbm.at[hqv[b]], qh[b], qsem[b])
      pltpu.async_copy(q_hbm.at[tqv[b]], qt[b], qsem[b])

    def wait_q(b):
      pltpu.make_async_copy(q_hbm.at[hqv[b]], qh[b], qsem[b]).wait()
      pltpu.make_async_copy(q_hbm.at[tqv[b]], qt[b], qsem[b]).wait()

    def wout_desc(k, b):
      off = base + k * B
      return pltpu.make_async_copy(wbuf[b], w_hbm.at[pl.ds(off, B)], wsem[b])

    issue_linear(0, 0)
    issue_linear(1, 1)
    wait_linear(0, 0)
    issue_q(0)

    def do_iter(k, b):
      nb = 1 - b
      wait_q(b)

      @pl.when(k + 1 < CH)
      def _():
        wait_linear(k + 1, nb)
        issue_q(nb)

      @pl.when(k >= 2)
      def _():
        wout_desc(k - 2, b).wait()

      for j in range(B // L):
        s = pl.ds(j * L, L)
        wv = qh[b][s] * qt[b][s]
        wbuf[b][s] = wv
        _scatter_max(smax_loc, hsc, vsc, headv[b][s], wv, lanes)

      pltpu.async_copy(wbuf[b], w_hbm.at[pl.ds(base + k * B, B)], wsem[b])

      @pl.when(k + 2 < CH)
      def _():
        issue_linear(k + 2, b)

    def pair(g, c):
      k0 = g * 2
      do_iter(k0, 0)
      do_iter(k0 + 1, 1)
      return c
    lax.fori_loop(0, CH // 2, pair, 0)
    if CH % 2:
      do_iter(CH - 1, 0)

    wout_desc(CH - 2, (CH - 2) % 2).wait()
    wout_desc(CH - 1, (CH - 1) % 2).wait()
    pltpu.sync_copy(smax_loc, smax_hbm.at[wid])

  mesh = plsc.VectorSubcoreMesh(core_axis_name="c", subcore_axis_name="s",
                                num_cores=NC, num_subcores=NS)
  return pl.kernel(
      body,
      out_type=[jax.ShapeDtypeStruct((E,), jnp.float32),
                jax.ShapeDtypeStruct((NW, N), jnp.float32)],
      mesh=mesh,
      compiler_params=pltpu.CompilerParams(needs_layout_passes=False),
      scratch_types=[
          [pltpu.VMEM((B,), jnp.int32)] * 2,
          [pltpu.VMEM((B,), jnp.int32)] * 2,
          [pltpu.VMEM((B,), jnp.int32)] * 2,
          [pltpu.VMEM((B,), jnp.float32)] * 2,
          [pltpu.VMEM((B,), jnp.float32)] * 2,
          [pltpu.VMEM((B,), jnp.float32)] * 2,
          pltpu.VMEM((N,), jnp.float32),
          pltpu.VMEM((128,), jnp.float32),
          pltpu.VMEM((128,), jnp.float32),
          [pltpu.SemaphoreType.DMA] * 2,
          [pltpu.SemaphoreType.DMA] * 2,
          [pltpu.SemaphoreType.DMA] * 2,
      ],
  )


def _make_phase_c(N, D, R, E):
  EP = E // NW
  CH = EP // B
  STRIPE = (N // NS) // 8 * 8
  TAIL = N - NS * STRIPE

  def body(emb_hbm, wt_hbm, head_hbm, tail_hbm, rel_hbm, we_hbm, smax_hbm,
           zrows_hbm, agg_hbm, sump_hbm,
           headv, tailv, relv, wv, shead, pbuf, rows,
           smax_loc, ssum_loc, hsc, vsc, wtab, agg_sh, lsem, gsem, ssem):
    cid = lax.axis_index("c")
    sid = lax.axis_index("s")
    wid = cid * NS + sid
    base = wid * EP
    lanes = lax.iota(jnp.int32, L)
    zeros = jnp.zeros((L,), jnp.float32)

    pltpu.sync_copy(smax_hbm, smax_loc)
    pltpu.sync_copy(wt_hbm, wtab)

    def zinit(i, c):
      ssum_loc[pl.ds(i * L, L)] = zeros
      return c
    lax.fori_loop(0, N // L, zinit, 0)

    pltpu.sync_copy(zrows_hbm, agg_sh.at[pl.ds(sid * STRIPE, STRIPE)])

    @pl.when(sid == 0)
    def _():
      pltpu.sync_copy(zrows_hbm.at[pl.ds(0, TAIL)],
                      agg_sh.at[pl.ds(NS * STRIPE, TAIL)])
    plsc.subcore_barrier()

    def issue_linear(k, b):
      off = base + k * B
      pltpu.async_copy(head_hbm.at[pl.ds(off, B)], headv[b], lsem[b])
      pltpu.async_copy(tail_hbm.at[pl.ds(off, B)], tailv[b], lsem[b])
      pltpu.async_copy(rel_hbm.at[pl.ds(off, B)], relv[b], lsem[b])
      pltpu.async_copy(we_hbm.at[pl.ds(off, B)], wv[b], lsem[b])

    def wait_linear(k, b):
      off = base + k * B
      pltpu.make_async_copy(head_hbm.at[pl.ds(off, B)], headv[b], lsem[b]).wait()
      pltpu.make_async_copy(tail_hbm.at[pl.ds(off, B)], tailv[b], lsem[b]).wait()
      pltpu.make_async_copy(rel_hbm.at[pl.ds(off, B)], relv[b], lsem[b]).wait()
      pltpu.make_async_copy(we_hbm.at[pl.ds(off, B)], wv[b], lsem[b]).wait()

    def scat_desc(b):
      return pltpu.make_async_copy(rows[b], agg_sh.at[shead[b]], ssem[b])

    issue_linear(0, 0)
    issue_linear(1, 1)
    wait_linear(0, 0)
    pltpu.async_copy(emb_hbm.at[tailv[0]], rows[0], gsem[0])

    def do_iter(k, b):
      nb = 1 - b
      pltpu.make_async_copy(emb_hbm.at[tailv[b]], rows[b], gsem[b]).wait()

      @pl.when(k + 1 < CH)
      def _():
        wait_linear(k + 1, nb)

        @pl.when(k >= 1)
        def _():
          scat_desc(nb).wait()
        pltpu.async_copy(emb_hbm.at[tailv[nb]], rows[nb], gsem[nb])

      for j in range(B // L):
        s = pl.ds(j * L, L)
        hv = headv[b][s]
        shead[b][s] = hv
        m = plsc.load_gather(smax_loc, [hv])
        p = jnp.exp(wv[b][s] - m)
        pbuf[s] = p
        _scatter_add(ssum_loc, hsc, vsc, hv, p, lanes)

      def edge(i, c2):
        isp = jnp.zeros((L,), jnp.int32) + i
        psp = plsc.load_gather(pbuf, [isp])
        rsp = plsc.load_gather(relv[b], [isp]).astype(jnp.int32)
        for jj in range(D // L):
          seg = pl.ds(jj * L, L)
          rows[b][i, seg] = psp * rows[b][i, seg] * plsc.load_gather(
              wtab, [rsp * D + jj * L + lanes])
        return c2
      lax.fori_loop(0, B, edge, 0)

      pltpu.async_copy(rows[b], agg_sh.at[shead[b]], ssem[b], add=True)

      @pl.when(k + 2 < CH)
      def _():
        issue_linear(k + 2, b)

    def pair(g, c):
      k0 = g * 2
      do_iter(k0, 0)
      do_iter(k0 + 1, 1)
      return c
    lax.fori_loop(0, CH // 2, pair, 0)
    if CH % 2:
      do_iter(CH - 1, 0)

    scat_desc((CH - 2) % 2).wait()
    scat_desc((CH - 1) % 2).wait()

    plsc.subcore_barrier()
    pltpu.sync_copy(agg_sh.at[pl.ds(sid * STRIPE, STRIPE)],
                    agg_hbm.at[pl.ds(cid * N + sid * STRIPE, STRIPE)])

    @pl.when(sid == 0)
    def _():
      pltpu.sync_copy(agg_sh.at[pl.ds(NS * STRIPE, TAIL)],
                      agg_hbm.at[pl.ds(cid * N + NS * STRIPE, TAIL)])
    pltpu.sync_copy(ssum_loc, sump_hbm.at[wid])

  mesh = plsc.VectorSubcoreMesh(core_axis_name="c", subcore_axis_name="s",
                                num_cores=NC, num_subcores=NS)
  return pl.kernel(
      body,
      out_type=[jax.ShapeDtypeStruct((NC * N, D), jnp.float32),
                jax.ShapeDtypeStruct((NW, N), jnp.float32)],
      mesh=mesh,
      compiler_params=pltpu.CompilerParams(needs_layout_passes=False),
      scratch_types=[
          [pltpu.VMEM((B,), jnp.int32)] * 2,
          [pltpu.VMEM((B,), jnp.int32)] * 2,
          [pltpu.VMEM((B,), jnp.float32)] * 2,
          [pltpu.VMEM((B,), jnp.float32)] * 2,
          [pltpu.VMEM((B,), jnp.int32)] * 2,
          pltpu.VMEM((B,), jnp.float32),
          [pltpu.VMEM((B, D), jnp.float32)] * 2,
          pltpu.VMEM((N,), jnp.float32),
          pltpu.VMEM((N,), jnp.float32),
          pltpu.VMEM((128,), jnp.float32),
          pltpu.VMEM((128,), jnp.float32),
          pltpu.VMEM((R * D,), jnp.float32),
          pltpu.VMEM_SHARED((N, D), jnp.float32),
          [pltpu.SemaphoreType.DMA] * 2,
          [pltpu.SemaphoreType.DMA] * 2,
          [pltpu.SemaphoreType.DMA] * 2,
      ],
  )


def _make_matmuls(U, N, D):
  BU = 256
  UB = U // BU

  def body(im_ref, emb_ref, uemb_ref, uout_ref, dout_ref):
    u = pl.program_id(0)
    im = im_ref[...]
    uout_ref[...] = lax.dot_general(im, emb_ref[...], (((1,), (0,)), ((), ())),
                                    precision=HIGH,
                                    preferred_element_type=jnp.float32)
    prod_d = lax.dot_general(im, uemb_ref[...], (((0,), (0,)), ((), ())),
                             precision=HIGH,
                             preferred_element_type=jnp.float32)

    @pl.when(u == 0)
    def _():
      dout_ref[...] = prod_d

    @pl.when(u != 0)
    def _():
      dout_ref[...] += prod_d

  return pl.pallas_call(
      body,
      grid=(UB,),
      in_specs=[
          pl.BlockSpec((BU, N), lambda u: (u, 0)),
          pl.BlockSpec((N, D), lambda u: (0, 0)),
          pl.BlockSpec((BU, D), lambda u: (u, 0)),
      ],
      out_specs=[
          pl.BlockSpec((BU, D), lambda u: (u, 0)),
          pl.BlockSpec((N, D), lambda u: (0, 0)),
      ],
      out_shape=[jax.ShapeDtypeStruct((U, D), jnp.float32),
                 jax.ShapeDtypeStruct((N, D), jnp.float32)],
  )


def _make_combine(N, D):
  def body(dense_ref, agg_ref, sum_ref, out_ref):
    s = jnp.sum(sum_ref[...], axis=0)
    a = agg_ref[0] + agg_ref[1]
    out_ref[...] = dense_ref[...] + a * (1.0 / (s + EPS))[:, None]

  return pl.pallas_call(
      body,
      out_shape=jax.ShapeDtypeStruct((N, D), jnp.float32),
  )


@jax.jit
def kernel(entity_emb, user_emb, edge_index, edge_type, interact_mat, weight):
  N, D = entity_emb.shape
  U = user_emb.shape[0]
  E = edge_index.shape[1]
  R = weight.shape[0]

  head = edge_index[0]
  tail = edge_index[1]
  rel = edge_type - 1

  q = _q_table(entity_emb, weight)
  hq_idx = head * R + rel
  tq_idx = tail * R + rel
  w_e, smax_parts = _make_phase_a(N, R, E)(q.reshape(-1), head, hq_idx, tq_idx)
  seg_max = _reduce_max(smax_parts)
  zrows = jnp.zeros(((N // NS) // 8 * 8, D), jnp.float32)
  agg_flat, sum_parts = _make_phase_c(N, D, R, E)(
      entity_emb, weight.reshape(-1), head, tail, rel.astype(jnp.float32),
      w_e, seg_max, zrows)
  agg_parts = agg_flat.reshape(NC, N, D)
  user_agg, dense = _make_matmuls(U, N, D)(interact_mat, entity_emb, user_emb)
  entity_agg = _make_combine(N, D)(dense, agg_parts, sum_parts)
  return entity_agg, user_agg


# speedup vs baseline: 10.5197x; 1.3159x over previous
import functools
import jax
import jax.numpy as jnp
from jax import lax
from jax.experimental import pallas as pl
from jax.experimental.pallas import tpu as pltpu
from jax.experimental.pallas import tpu_sc as plsc

NC, NS, L = 2, 16, 16
NW = NC * NS
B = 80
EPS = 1e-16
HIGH = lax.Precision.HIGHEST


def _dedup_combine(hsc, vsc, hv, vv, lanes, is_add):
  hvf = hv.astype(jnp.float32)
  hsc[pl.ds(0, L)] = hvf
  vsc[pl.ds(0, L)] = vv
  acc = vv
  first = jnp.ones((L,), jnp.bool_)
  for s in range(1, L):
    idx = (lanes + s) & (L - 1)
    hr = plsc.load_gather(hsc, [idx])
    vr = plsc.load_gather(vsc, [idx])
    same = hr == hvf
    comb = acc + vr if is_add else jnp.maximum(acc, vr)
    acc = jnp.where(same, comb, acc)
    first = first & jnp.logical_not(same & (idx < lanes))
  return acc, first


def _scatter_max(tab, hsc, vsc, hv, vv, lanes):
  acc, first = _dedup_combine(hsc, vsc, hv, vv, lanes, False)
  cur = plsc.load_gather(tab, [hv])
  plsc.store_scatter(tab, [hv], jnp.maximum(acc, cur), mask=first)


def _scatter_add(tab, hsc, vsc, hv, vv, lanes):
  acc, first = _dedup_combine(hsc, vsc, hv, vv, lanes, True)
  cur = plsc.load_gather(tab, [hv])
  plsc.store_scatter(tab, [hv], cur + acc, mask=first)


def _q_body(emb_ref, w_ref, q_ref):
  e2 = emb_ref[...] * emb_ref[...]
  w2 = w_ref[...] * w_ref[...]
  q_ref[...] = lax.dot_general(e2, w2, (((1,), (1,)), ((), ())),
                               precision=HIGH,
                               preferred_element_type=jnp.float32)


def _q_table(entity_emb, weight):
  N, _ = entity_emb.shape
  R = weight.shape[0]
  return pl.pallas_call(
      _q_body,
      out_shape=jax.ShapeDtypeStruct((N, R), jnp.float32),
  )(entity_emb, weight)


def _maxred_body(parts_ref, out_ref):
  out_ref[...] = jnp.max(parts_ref[...], axis=0)


def _reduce_max(parts):
  _, N = parts.shape
  return pl.pallas_call(
      _maxred_body,
      out_shape=jax.ShapeDtypeStruct((N,), jnp.float32),
  )(parts)


def _make_phase_a(N, R, E):
  EP = E // NW
  CH = EP // B

  def body(q_hbm, head_hbm, hq_hbm, tq_hbm, w_hbm, smax_hbm,
           headv, hqv, tqv, qh, qt, wbuf, smax_loc, hsc, vsc,
           lsem, qsem, wsem):
    cid = lax.axis_index("c")
    sid = lax.axis_index("s")
    wid = cid * NS + sid
    base = wid * EP
    lanes = lax.iota(jnp.int32, L)
    zeros = jnp.zeros((L,), jnp.float32)

    def zinit(i, c):
      smax_loc[pl.ds(i * L, L)] = zeros
      return c
    lax.fori_loop(0, N // L, zinit, 0)

    def issue_linear(k, b):
      off = base + k * B
      pltpu.async_copy(head_hbm.at[pl.ds(off, B)], headv[b], lsem[b])
      pltpu.async_copy(hq_hbm.at[pl.ds(off, B)], hqv[b], lsem[b])
      pltpu.async_copy(tq_hbm.at[pl.ds(off, B)], tqv[b], lsem[b])

    def wait_linear(k, b):
      off = base + k * B
      pltpu.make_async_copy(head_hbm.at[pl.ds(off, B)], headv[b], lsem[b]).wait()
      pltpu.make_async_copy(hq_hbm.at[pl.ds(off, B)], hqv[b], lsem[b]).wait()
      pltpu.make_async_copy(tq_hbm.at[pl.ds(off, B)], tqv[b], lsem[b]).wait()

    def issue_q(b):
      pltpu.async_copy(q_hbm.at[hqv[b]], qh[b], qsem[b])
      pltpu.async_copy(q_hbm.at[tqv[b]], qt[b], qsem[b])

    def wait_q(b):
      pltpu.make_async_copy(q_hbm.at[hqv[b]], qh[b], qsem[b]).wait()
      pltpu.make_async_copy(q_hbm.at[tqv[b]], qt[b], qsem[b]).wait()

    def wout_desc(k, b):
      off = base + k * B
      return pltpu.make_async_copy(wbuf[b], w_hbm.at[pl.ds(off, B)], wsem[b])

    issue_linear(0, 0)
    issue_linear(1, 1)
    wait_linear(0, 0)
    issue_q(0)

    def do_iter(k, b):
      nb = 1 - b
      wait_q(b)

      @pl.when(k + 1 < CH)
      def _():
        wait_linear(k + 1, nb)
        issue_q(nb)

      @pl.when(k >= 2)
      def _():
        wout_desc(k - 2, b).wait()

      for j in range(B // L):
        s = pl.ds(j * L, L)
        wv = qh[b][s] * qt[b][s]
        wbuf[b][s] = wv
        _scatter_max(smax_loc, hsc, vsc, headv[b][s], wv, lanes)

      pltpu.async_copy(wbuf[b], w_hbm.at[pl.ds(base + k * B, B)], wsem[b])

      @pl.when(k + 2 < CH)
      def _():
        issue_linear(k + 2, b)

    def pair(g, c):
      k0 = g * 2
      do_iter(k0, 0)
      do_iter(k0 + 1, 1)
      return c
    lax.fori_loop(0, CH // 2, pair, 0)
    if CH % 2:
      do_iter(CH - 1, 0)

    wout_desc(CH - 2, (CH - 2) % 2).wait()
    wout_desc(CH - 1, (CH - 1) % 2).wait()
    pltpu.sync_copy(smax_loc, smax_hbm.at[wid])

  mesh = plsc.VectorSubcoreMesh(core_axis_name="c", subcore_axis_name="s",
                                num_cores=NC, num_subcores=NS)
  return pl.kernel(
      body,
      out_type=[jax.ShapeDtypeStruct((E,), jnp.float32),
                jax.ShapeDtypeStruct((NW, N), jnp.float32)],
      mesh=mesh,
      compiler_params=pltpu.CompilerParams(needs_layout_passes=False),
      scratch_types=[
          [pltpu.VMEM((B,), jnp.int32)] * 2,
          [pltpu.VMEM((B,), jnp.int32)] * 2,
          [pltpu.VMEM((B,), jnp.int32)] * 2,
          [pltpu.VMEM((B,), jnp.float32)] * 2,
          [pltpu.VMEM((B,), jnp.float32)] * 2,
          [pltpu.VMEM((B,), jnp.float32)] * 2,
          pltpu.VMEM((N,), jnp.float32),
          pltpu.VMEM((128,), jnp.float32),
          pltpu.VMEM((128,), jnp.float32),
          [pltpu.SemaphoreType.DMA] * 2,
          [pltpu.SemaphoreType.DMA] * 2,
          [pltpu.SemaphoreType.DMA] * 2,
      ],
  )


def _make_mtable(N, R, D):
  BN = 400

  def body(emb_ref, w_ref, out_ref):
    out_ref[...] = emb_ref[...][:, None, :] * w_ref[...][None, :, :]

  return pl.pallas_call(
      body,
      grid=(N // BN,),
      in_specs=[
          pl.BlockSpec((BN, D), lambda n: (n, 0)),
          pl.BlockSpec((R, D), lambda n: (0, 0)),
      ],
      out_specs=pl.BlockSpec((BN, R, D), lambda n: (n, 0, 0)),
      out_shape=jax.ShapeDtypeStruct((N, R, D), jnp.float32),
  )


def _make_phase_c(N, D, R, E):
  EP = E // NW
  CH = EP // B
  STRIPE = (N // NS) // 8 * 8
  TAIL = N - NS * STRIPE

  def body(m_hbm, head_hbm, tq_hbm, we_hbm, smax_hbm, zrows_hbm, zsum_hbm,
           agg_hbm, sump_hbm,
           headv, tqv, wv, shead, pbuf, rows,
           smax_loc, agg_sh, ssum_sh, lsem, gsem, ssem):
    cid = lax.axis_index("c")
    sid = lax.axis_index("s")
    base = (cid * NS + sid) * EP
    lanes = lax.iota(jnp.int32, L)

    pltpu.sync_copy(smax_hbm, smax_loc)

    pltpu.sync_copy(zrows_hbm, agg_sh.at[pl.ds(sid * STRIPE, STRIPE)])

    @pl.when(sid == 0)
    def _():
      pltpu.sync_copy(zrows_hbm.at[pl.ds(0, TAIL)],
                      agg_sh.at[pl.ds(NS * STRIPE, TAIL)])
      pltpu.sync_copy(zsum_hbm, ssum_sh)
    plsc.subcore_barrier()

    def issue_linear(k, b):
      off = base + k * B
      pltpu.async_copy(head_hbm.at[pl.ds(off, B)], headv[b], lsem[b])
      pltpu.async_copy(tq_hbm.at[pl.ds(off, B)], tqv[b], lsem[b])
      pltpu.async_copy(we_hbm.at[pl.ds(off, B)], wv[b], lsem[b])

    def wait_linear(k, b):
      off = base + k * B
      pltpu.make_async_copy(head_hbm.at[pl.ds(off, B)], headv[b], lsem[b]).wait()
      pltpu.make_async_copy(tq_hbm.at[pl.ds(off, B)], tqv[b], lsem[b]).wait()
      pltpu.make_async_copy(we_hbm.at[pl.ds(off, B)], wv[b], lsem[b]).wait()

    def scat_waits(b):
      pltpu.make_async_copy(rows[b], agg_sh.at[shead[b]], ssem[b]).wait()
      pltpu.make_async_copy(pbuf[b], ssum_sh.at[shead[b]], ssem[b]).wait()

    issue_linear(0, 0)
    issue_linear(1, 1)
    wait_linear(0, 0)
    pltpu.async_copy(m_hbm.at[tqv[0]], rows[0], gsem[0])

    def do_iter(k, b):
      nb = 1 - b
      pltpu.make_async_copy(m_hbm.at[tqv[b]], rows[b], gsem[b]).wait()

      @pl.when(k + 1 < CH)
      def _():
        wait_linear(k + 1, nb)

        @pl.when(k >= 1)
        def _():
          scat_waits(nb)
        pltpu.async_copy(m_hbm.at[tqv[nb]], rows[nb], gsem[nb])

      for j in range(B // L):
        s = pl.ds(j * L, L)
        hv = headv[b][s]
        shead[b][s] = hv
        m = plsc.load_gather(smax_loc, [hv])
        pbuf[b][s] = jnp.exp(wv[b][s] - m)

      def edge(i, c2):
        for u in range(2):
          iu = i * 2 + u
          isp = jnp.zeros((L,), jnp.int32) + iu
          psp = plsc.load_gather(pbuf[b], [isp])
          for jj in range(D // L):
            seg = pl.ds(jj * L, L)
            rows[b][iu, seg] = psp * rows[b][iu, seg]
        return c2
      lax.fori_loop(0, B // 2, edge, 0)

      pltpu.async_copy(rows[b], agg_sh.at[shead[b]], ssem[b], add=True)
      pltpu.async_copy(pbuf[b], ssum_sh.at[shead[b]], ssem[b], add=True)

      @pl.when(k + 2 < CH)
      def _():
        issue_linear(k + 2, b)

    def pair(g, c):
      k0 = g * 2
      do_iter(k0, 0)
      do_iter(k0 + 1, 1)
      return c
    lax.fori_loop(0, CH // 2, pair, 0)
    if CH % 2:
      do_iter(CH - 1, 0)

    scat_waits((CH - 2) % 2)
    scat_waits((CH - 1) % 2)

    plsc.subcore_barrier()
    pltpu.sync_copy(agg_sh.at[pl.ds(sid * STRIPE, STRIPE)],
                    agg_hbm.at[pl.ds(cid * N + sid * STRIPE, STRIPE)])

    @pl.when(sid == 0)
    def _():
      pltpu.sync_copy(agg_sh.at[pl.ds(NS * STRIPE, TAIL)],
                      agg_hbm.at[pl.ds(cid * N + NS * STRIPE, TAIL)])
      pltpu.sync_copy(ssum_sh, sump_hbm.at[cid])

  mesh = plsc.VectorSubcoreMesh(core_axis_name="c", subcore_axis_name="s",
                                num_cores=NC, num_subcores=NS)
  return pl.kernel(
      body,
      out_type=[jax.ShapeDtypeStruct((NC * N, D), jnp.float32),
                jax.ShapeDtypeStruct((NC, N), jnp.float32)],
      mesh=mesh,
      compiler_params=pltpu.CompilerParams(needs_layout_passes=False),
      scratch_types=[
          [pltpu.VMEM((B,), jnp.int32)] * 2,
          [pltpu.VMEM((B,), jnp.int32)] * 2,
          [pltpu.VMEM((B,), jnp.float32)] * 2,
          [pltpu.VMEM((B,), jnp.int32)] * 2,
          [pltpu.VMEM((B,), jnp.float32)] * 2,
          [pltpu.VMEM((B, D), jnp.float32)] * 2,
          pltpu.VMEM((N,), jnp.float32),
          pltpu.VMEM_SHARED((N, D), jnp.float32),
          pltpu.VMEM_SHARED((N,), jnp.float32),
          [pltpu.SemaphoreType.DMA] * 2,
          [pltpu.SemaphoreType.DMA] * 2,
          [pltpu.SemaphoreType.DMA] * 2,
      ],
  )


def _make_matmuls(U, N, D):
  BU = 256
  UB = U // BU

  def body(im_ref, emb_ref, uemb_ref, uout_ref, dout_ref):
    u = pl.program_id(0)
    im = im_ref[...]
    uout_ref[...] = lax.dot_general(im, emb_ref[...], (((1,), (0,)), ((), ())),
                                    precision=HIGH,
                                    preferred_element_type=jnp.float32)
    prod_d = lax.dot_general(im, uemb_ref[...], (((0,), (0,)), ((), ())),
                             precision=HIGH,
                             preferred_element_type=jnp.float32)

    @pl.when(u == 0)
    def _():
      dout_ref[...] = prod_d

    @pl.when(u != 0)
    def _():
      dout_ref[...] += prod_d

  return pl.pallas_call(
      body,
      grid=(UB,),
      in_specs=[
          pl.BlockSpec((BU, N), lambda u: (u, 0)),
          pl.BlockSpec((N, D), lambda u: (0, 0)),
          pl.BlockSpec((BU, D), lambda u: (u, 0)),
      ],
      out_specs=[
          pl.BlockSpec((BU, D), lambda u: (u, 0)),
          pl.BlockSpec((N, D), lambda u: (0, 0)),
      ],
      out_shape=[jax.ShapeDtypeStruct((U, D), jnp.float32),
                 jax.ShapeDtypeStruct((N, D), jnp.float32)],
  )


def _make_combine(N, D):
  def body(dense_ref, agg_ref, sum_ref, out_ref):
    s = sum_ref[0] + sum_ref[1]
    a = agg_ref[0] + agg_ref[1]
    out_ref[...] = dense_ref[...] + a * (1.0 / (s + EPS))[:, None]

  return pl.pallas_call(
      body,
      out_shape=jax.ShapeDtypeStruct((N, D), jnp.float32),
  )


@jax.jit
def kernel(entity_emb, user_emb, edge_index, edge_type, interact_mat, weight):
  N, D = entity_emb.shape
  U = user_emb.shape[0]
  E = edge_index.shape[1]
  R = weight.shape[0]

  head = edge_index[0]
  tail = edge_index[1]
  rel = edge_type - 1

  q = _q_table(entity_emb, weight)
  hq_idx = head * R + rel
  tq_idx = tail * R + rel
  w_e, smax_parts = _make_phase_a(N, R, E)(q.reshape(-1), head, hq_idx, tq_idx)
  seg_max = _reduce_max(smax_parts)
  zrows = jnp.zeros(((N // NS) // 8 * 8, D), jnp.float32)
  zsum = jnp.zeros((N,), jnp.float32)
  m_table = _make_mtable(N, R, D)(entity_emb, weight).reshape(N * R, D)
  agg_flat, sum_parts = _make_phase_c(N, D, R, E)(
      m_table, head, tq_idx, w_e, seg_max, zrows, zsum)
  agg_parts = agg_flat.reshape(NC, N, D)
  user_agg, dense = _make_matmuls(U, N, D)(interact_mat, entity_emb, user_emb)
  entity_agg = _make_combine(N, D)(dense, agg_parts, sum_parts)
  return entity_agg, user_agg


# speedup vs baseline: 15.3444x; 1.4586x over previous
import functools
import jax
import jax.numpy as jnp
from jax import lax
from jax.experimental import pallas as pl
from jax.experimental.pallas import tpu as pltpu
from jax.experimental.pallas import tpu_sc as plsc

NC, NS, L = 2, 16, 16
NW = NC * NS
B = 80
EPS = 1e-16
HIGH = lax.Precision.HIGHEST


def _dedup_combine(hsc, vsc, hv, vv, lanes, is_add):
  hvf = hv.astype(jnp.float32)
  hsc[pl.ds(0, L)] = hvf
  vsc[pl.ds(0, L)] = vv
  acc = vv
  first = jnp.ones((L,), jnp.bool_)
  for s in range(1, L):
    idx = (lanes + s) & (L - 1)
    hr = plsc.load_gather(hsc, [idx])
    vr = plsc.load_gather(vsc, [idx])
    same = hr == hvf
    comb = acc + vr if is_add else jnp.maximum(acc, vr)
    acc = jnp.where(same, comb, acc)
    first = first & jnp.logical_not(same & (idx < lanes))
  return acc, first


def _scatter_max(tab, hsc, vsc, hv, vv, lanes):
  acc, first = _dedup_combine(hsc, vsc, hv, vv, lanes, False)
  cur = plsc.load_gather(tab, [hv])
  plsc.store_scatter(tab, [hv], jnp.maximum(acc, cur), mask=first)


def _scatter_add(tab, hsc, vsc, hv, vv, lanes):
  acc, first = _dedup_combine(hsc, vsc, hv, vv, lanes, True)
  cur = plsc.load_gather(tab, [hv])
  plsc.store_scatter(tab, [hv], cur + acc, mask=first)


def _q_body(emb_ref, w_ref, q_ref):
  e2 = emb_ref[...] * emb_ref[...]
  w2 = w_ref[...] * w_ref[...]
  q_ref[...] = lax.dot_general(e2, w2, (((1,), (1,)), ((), ())),
                               precision=HIGH,
                               preferred_element_type=jnp.float32)


def _q_table(entity_emb, weight):
  N, _ = entity_emb.shape
  R = weight.shape[0]
  return pl.pallas_call(
      _q_body,
      out_shape=jax.ShapeDtypeStruct((N, R), jnp.float32),
  )(entity_emb, weight)


def _maxred_body(parts_ref, out_ref):
  out_ref[...] = jnp.max(parts_ref[...], axis=0)


def _reduce_max(parts):
  _, N = parts.shape
  return pl.pallas_call(
      _maxred_body,
      out_shape=jax.ShapeDtypeStruct((N,), jnp.float32),
  )(parts)


def _make_phase_a(N, R, E):
  EP = E // NW
  CH = EP // B

  def body(q_hbm, head_hbm, hq_hbm, tq_hbm, w_hbm, smax_hbm,
           headv, hqv, tqv, qh, qt, wbuf, smax_loc, hsc, vsc,
           lsem, qsem, wsem):
    cid = lax.axis_index("c")
    sid = lax.axis_index("s")
    wid = cid * NS + sid
    base = wid * EP
    lanes = lax.iota(jnp.int32, L)
    zeros = jnp.zeros((L,), jnp.float32)

    def zinit(i, c):
      smax_loc[pl.ds(i * L, L)] = zeros
      return c
    lax.fori_loop(0, N // L, zinit, 0)

    def issue_linear(k, b):
      off = base + k * B
      pltpu.async_copy(head_hbm.at[pl.ds(off, B)], headv[b], lsem[b])
      pltpu.async_copy(hq_hbm.at[pl.ds(off, B)], hqv[b], lsem[b])
      pltpu.async_copy(tq_hbm.at[pl.ds(off, B)], tqv[b], lsem[b])

    def wait_linear(k, b):
      off = base + k * B
      pltpu.make_async_copy(head_hbm.at[pl.ds(off, B)], headv[b], lsem[b]).wait()
      pltpu.make_async_copy(hq_hbm.at[pl.ds(off, B)], hqv[b], lsem[b]).wait()
      pltpu.make_async_copy(tq_hbm.at[pl.ds(off, B)], tqv[b], lsem[b]).wait()

    def issue_q(b):
      pltpu.async_copy(q_hbm.at[hqv[b]], qh[b], qsem[b])
      pltpu.async_copy(q_hbm.at[tqv[b]], qt[b], qsem[b])

    def wait_q(b):
      pltpu.make_async_copy(q_hbm.at[hqv[b]], qh[b], qsem[b]).wait()
      pltpu.make_async_copy(q_hbm.at[tqv[b]], qt[b], qsem[b]).wait()

    def wout_desc(k, b):
      off = base + k * B
      return pltpu.make_async_copy(wbuf[b], w_hbm.at[pl.ds(off, B)], wsem[b])

    issue_linear(0, 0)
    issue_linear(1, 1)
    wait_linear(0, 0)
    issue_q(0)

    def do_iter(k, b):
      nb = 1 - b
      wait_q(b)

      @pl.when(k + 1 < CH)
      def _():
        wait_linear(k + 1, nb)
        issue_q(nb)

      @pl.when(k >= 2)
      def _():
        wout_desc(k - 2, b).wait()

      for j in range(B // L):
        s = pl.ds(j * L, L)
        wv = qh[b][s] * qt[b][s]
        wbuf[b][s] = wv
        _scatter_max(smax_loc, hsc, vsc, headv[b][s], wv, lanes)

      pltpu.async_copy(wbuf[b], w_hbm.at[pl.ds(base + k * B, B)], wsem[b])

      @pl.when(k + 2 < CH)
      def _():
        issue_linear(k + 2, b)

    def pair(g, c):
      k0 = g * 2
      do_iter(k0, 0)
      do_iter(k0 + 1, 1)
      return c
    lax.fori_loop(0, CH // 2, pair, 0)
    if CH % 2:
      do_iter(CH - 1, 0)

    wout_desc(CH - 2, (CH - 2) % 2).wait()
    wout_desc(CH - 1, (CH - 1) % 2).wait()
    pltpu.sync_copy(smax_loc, smax_hbm.at[wid])

  mesh = plsc.VectorSubcoreMesh(core_axis_name="c", subcore_axis_name="s",
                                num_cores=NC, num_subcores=NS)
  return pl.kernel(
      body,
      out_type=[jax.ShapeDtypeStruct((E,), jnp.float32),
                jax.ShapeDtypeStruct((NW, N), jnp.float32)],
      mesh=mesh,
      compiler_params=pltpu.CompilerParams(needs_layout_passes=False),
      scratch_types=[
          [pltpu.VMEM((B,), jnp.int32)] * 2,
          [pltpu.VMEM((B,), jnp.int32)] * 2,
          [pltpu.VMEM((B,), jnp.int32)] * 2,
          [pltpu.VMEM((B,), jnp.float32)] * 2,
          [pltpu.VMEM((B,), jnp.float32)] * 2,
          [pltpu.VMEM((B,), jnp.float32)] * 2,
          pltpu.VMEM((N,), jnp.float32),
          pltpu.VMEM((128,), jnp.float32),
          pltpu.VMEM((128,), jnp.float32),
          [pltpu.SemaphoreType.DMA] * 2,
          [pltpu.SemaphoreType.DMA] * 2,
          [pltpu.SemaphoreType.DMA] * 2,
      ],
  )


def _make_mtable(N, R, D):
  BN = 400

  def body(emb_ref, w_ref, out_ref):
    out_ref[...] = emb_ref[...][:, None, :] * w_ref[...][None, :, :]

  return pl.pallas_call(
      body,
      grid=(N // BN,),
      in_specs=[
          pl.BlockSpec((BN, D), lambda n: (n, 0)),
          pl.BlockSpec((R, D), lambda n: (0, 0)),
      ],
      out_specs=pl.BlockSpec((BN, R, D), lambda n: (n, 0, 0)),
      out_shape=jax.ShapeDtypeStruct((N, R, D), jnp.float32),
  )


def _make_phase_c(N, D, R, E):
  EP = E // NW
  CH = EP // B
  STRIPE = (N // NS) // 8 * 8
  TAIL = N - NS * STRIPE

  def body(m_hbm, head_hbm, tq_hbm, we_hbm, smax_hbm, zrows_hbm, zsum_hbm,
           agg_hbm, sump_hbm,
           headv, tqv, wv, shead, pbuf, rows,
           smax_loc, agg_sh, ssum_sh, lsem, gsem, ssem):
    cid = lax.axis_index("c")
    sid = lax.axis_index("s")
    base = (cid * NS + sid) * EP
    lanes = lax.iota(jnp.int32, L)

    pltpu.sync_copy(smax_hbm, smax_loc)

    pltpu.sync_copy(zrows_hbm, agg_sh.at[pl.ds(sid * STRIPE, STRIPE)])

    @pl.when(sid == 0)
    def _():
      pltpu.sync_copy(zrows_hbm.at[pl.ds(0, TAIL)],
                      agg_sh.at[pl.ds(NS * STRIPE, TAIL)])
      pltpu.sync_copy(zsum_hbm, ssum_sh)
    plsc.subcore_barrier()

    def issue_linear(k, b):
      off = base + k * B
      pltpu.async_copy(head_hbm.at[pl.ds(off, B)], headv[b], lsem[b])
      pltpu.async_copy(tq_hbm.at[pl.ds(off, B)], tqv[b], lsem[b])
      pltpu.async_copy(we_hbm.at[pl.ds(off, B)], wv[b], lsem[b])

    def wait_linear(k, b):
      off = base + k * B
      pltpu.make_async_copy(head_hbm.at[pl.ds(off, B)], headv[b], lsem[b]).wait()
      pltpu.make_async_copy(tq_hbm.at[pl.ds(off, B)], tqv[b], lsem[b]).wait()
      pltpu.make_async_copy(we_hbm.at[pl.ds(off, B)], wv[b], lsem[b]).wait()

    def scat_waits(b):
      pltpu.make_async_copy(rows[b], agg_sh.at[shead[b]], ssem[b]).wait()
      pltpu.make_async_copy(pbuf[b], ssum_sh.at[shead[b]], ssem[b]).wait()

    issue_linear(0, 0)
    issue_linear(1, 1)
    wait_linear(0, 0)
    pltpu.async_copy(m_hbm.at[tqv[0]], rows[0], gsem[0])

    def do_iter(k, b):
      nb = 1 - b
      pltpu.make_async_copy(m_hbm.at[tqv[b]], rows[b], gsem[b]).wait()

      @pl.when(k + 1 < CH)
      def _():
        wait_linear(k + 1, nb)

        @pl.when(k >= 1)
        def _():
          scat_waits(nb)
        pltpu.async_copy(m_hbm.at[tqv[nb]], rows[nb], gsem[nb])

      for j in range(B // L):
        s = pl.ds(j * L, L)
        hv = headv[b][s]
        shead[b][s] = hv
        m = plsc.load_gather(smax_loc, [hv])
        pbuf[b][s] = jnp.exp(wv[b][s] - m)

      def edge(i, c2):
        for u in range(2):
          iu = i * 2 + u
          isp = jnp.zeros((L,), jnp.int32) + iu
          psp = plsc.load_gather(pbuf[b], [isp])
          for jj in range(D // L):
            seg = pl.ds(jj * L, L)
            rows[b][iu, seg] = psp * rows[b][iu, seg]
        return c2
      lax.fori_loop(0, B // 2, edge, 0)

      pltpu.async_copy(rows[b], agg_sh.at[shead[b]], ssem[b], add=True)
      pltpu.async_copy(pbuf[b], ssum_sh.at[shead[b]], ssem[b], add=True)

      @pl.when(k + 2 < CH)
      def _():
        issue_linear(k + 2, b)

    def pair(g, c):
      k0 = g * 2
      do_iter(k0, 0)
      do_iter(k0 + 1, 1)
      return c
    lax.fori_loop(0, CH // 2, pair, 0)
    if CH % 2:
      do_iter(CH - 1, 0)

    scat_waits((CH - 2) % 2)
    scat_waits((CH - 1) % 2)

    plsc.subcore_barrier()
    pltpu.sync_copy(agg_sh.at[pl.ds(sid * STRIPE, STRIPE)],
                    agg_hbm.at[pl.ds(cid * N + sid * STRIPE, STRIPE)])

    @pl.when(sid == 0)
    def _():
      pltpu.sync_copy(agg_sh.at[pl.ds(NS * STRIPE, TAIL)],
                      agg_hbm.at[pl.ds(cid * N + NS * STRIPE, TAIL)])
      pltpu.sync_copy(ssum_sh, sump_hbm.at[cid])

  mesh = plsc.VectorSubcoreMesh(core_axis_name="c", subcore_axis_name="s",
                                num_cores=NC, num_subcores=NS)
  return pl.kernel(
      body,
      out_type=[jax.ShapeDtypeStruct((NC * N, D), jnp.float32),
                jax.ShapeDtypeStruct((NC, N), jnp.float32)],
      mesh=mesh,
      compiler_params=pltpu.CompilerParams(needs_layout_passes=False),
      scratch_types=[
          [pltpu.VMEM((B,), jnp.int32)] * 2,
          [pltpu.VMEM((B,), jnp.int32)] * 2,
          [pltpu.VMEM((B,), jnp.float32)] * 2,
          [pltpu.VMEM((B,), jnp.int32)] * 2,
          [pltpu.VMEM((B,), jnp.float32)] * 2,
          [pltpu.VMEM((B, D), jnp.float32)] * 2,
          pltpu.VMEM((N,), jnp.float32),
          pltpu.VMEM_SHARED((N, D), jnp.float32),
          pltpu.VMEM_SHARED((N,), jnp.float32),
          [pltpu.SemaphoreType.DMA] * 2,
          [pltpu.SemaphoreType.DMA] * 2,
          [pltpu.SemaphoreType.DMA] * 2,
      ],
  )


def _make_matmuls(U, N, D):
  BU = 256
  UB = U // BU

  def body(im_ref, emb_ref, uemb_ref, uout_ref, dout_ref):
    u = pl.program_id(0)
    im = im_ref[...]
    uout_ref[...] = lax.dot_general(im, emb_ref[...], (((1,), (0,)), ((), ())),
                                    preferred_element_type=jnp.float32)
    prod_d = lax.dot_general(im, uemb_ref[...], (((0,), (0,)), ((), ())),
                             preferred_element_type=jnp.float32)

    @pl.when(u == 0)
    def _():
      dout_ref[...] = prod_d

    @pl.when(u != 0)
    def _():
      dout_ref[...] += prod_d

  return pl.pallas_call(
      body,
      grid=(UB,),
      in_specs=[
          pl.BlockSpec((BU, N), lambda u: (u, 0)),
          pl.BlockSpec((N, D), lambda u: (0, 0)),
          pl.BlockSpec((BU, D), lambda u: (u, 0)),
      ],
      out_specs=[
          pl.BlockSpec((BU, D), lambda u: (u, 0)),
          pl.BlockSpec((N, D), lambda u: (0, 0)),
      ],
      out_shape=[jax.ShapeDtypeStruct((U, D), jnp.float32),
                 jax.ShapeDtypeStruct((N, D), jnp.float32)],
  )


def _make_combine(N, D):
  def body(dense_ref, agg_ref, sum_ref, out_ref):
    s = sum_ref[0] + sum_ref[1]
    a = agg_ref[0] + agg_ref[1]
    out_ref[...] = dense_ref[...] + a * (1.0 / (s + EPS))[:, None]

  return pl.pallas_call(
      body,
      out_shape=jax.ShapeDtypeStruct((N, D), jnp.float32),
  )


@jax.jit
def kernel(entity_emb, user_emb, edge_index, edge_type, interact_mat, weight):
  N, D = entity_emb.shape
  U = user_emb.shape[0]
  E = edge_index.shape[1]
  R = weight.shape[0]

  head = edge_index[0]
  tail = edge_index[1]
  rel = edge_type - 1

  q = _q_table(entity_emb, weight)
  hq_idx = head * R + rel
  tq_idx = tail * R + rel
  w_e, smax_parts = _make_phase_a(N, R, E)(q.reshape(-1), head, hq_idx, tq_idx)
  seg_max = _reduce_max(smax_parts)
  zrows = jnp.zeros(((N // NS) // 8 * 8, D), jnp.float32)
  zsum = jnp.zeros((N,), jnp.float32)
  m_table = _make_mtable(N, R, D)(entity_emb, weight).reshape(N * R, D)
  agg_flat, sum_parts = _make_phase_c(N, D, R, E)(
      m_table, head, tq_idx, w_e, seg_max, zrows, zsum)
  agg_parts = agg_flat.reshape(NC, N, D)
  user_agg, dense = _make_matmuls(U, N, D)(interact_mat, entity_emb, user_emb)
  entity_agg = _make_combine(N, D)(dense, agg_parts, sum_parts)
  return entity_agg, user_agg


# speedup vs baseline: 17.4204x; 1.1353x over previous
import functools
import jax
import jax.numpy as jnp
from jax import lax
from jax.experimental import pallas as pl
from jax.experimental.pallas import tpu as pltpu
from jax.experimental.pallas import tpu_sc as plsc

NC, NS, L = 2, 16, 16
NW = NC * NS
B = 80
EPS = 1e-16
HIGH = lax.Precision.HIGHEST


def _dedup_combine(hsc, vsc, hv, vv, lanes, is_add):
  hvf = hv.astype(jnp.float32)
  hsc[pl.ds(0, L)] = hvf
  vsc[pl.ds(0, L)] = vv
  acc = vv
  first = jnp.ones((L,), jnp.bool_)
  for s in range(1, L):
    idx = (lanes + s) & (L - 1)
    hr = plsc.load_gather(hsc, [idx])
    vr = plsc.load_gather(vsc, [idx])
    same = hr == hvf
    comb = acc + vr if is_add else jnp.maximum(acc, vr)
    acc = jnp.where(same, comb, acc)
    first = first & jnp.logical_not(same & (idx < lanes))
  return acc, first


def _scatter_max(tab, hsc, vsc, hv, vv, lanes):
  acc, first = _dedup_combine(hsc, vsc, hv, vv, lanes, False)
  cur = plsc.load_gather(tab, [hv])
  plsc.store_scatter(tab, [hv], jnp.maximum(acc, cur), mask=first)


def _scatter_add(tab, hsc, vsc, hv, vv, lanes):
  acc, first = _dedup_combine(hsc, vsc, hv, vv, lanes, True)
  cur = plsc.load_gather(tab, [hv])
  plsc.store_scatter(tab, [hv], cur + acc, mask=first)


def _q_body(emb_ref, w_ref, q_ref):
  e2 = emb_ref[...] * emb_ref[...]
  w2 = w_ref[...] * w_ref[...]
  q_ref[...] = lax.dot_general(e2, w2, (((1,), (1,)), ((), ())),
                               precision=HIGH,
                               preferred_element_type=jnp.float32)


def _q_table(entity_emb, weight):
  N, _ = entity_emb.shape
  R = weight.shape[0]
  return pl.pallas_call(
      _q_body,
      out_shape=jax.ShapeDtypeStruct((N, R), jnp.float32),
  )(entity_emb, weight)


def _maxred_body(parts_ref, out_ref):
  out_ref[...] = jnp.max(parts_ref[...], axis=0)


def _reduce_max(parts):
  _, N = parts.shape
  return pl.pallas_call(
      _maxred_body,
      out_shape=jax.ShapeDtypeStruct((N,), jnp.float32),
  )(parts)


def _make_phase_a(N, R, E):
  EP = E // NW
  CH = EP // B
  NB = 4

  def body(q_hbm, head_hbm, hq_hbm, tq_hbm, w_hbm, smax_hbm,
           headv, hqv, tqv, qh, qt, wbuf, smax_loc, hsc, vsc,
           lsem, qsem, wsem):
    cid = lax.axis_index("c")
    sid = lax.axis_index("s")
    wid = cid * NS + sid
    base = wid * EP
    lanes = lax.iota(jnp.int32, L)
    zeros = jnp.zeros((L,), jnp.float32)

    def zinit(i, c):
      smax_loc[pl.ds(i * L, L)] = zeros
      return c
    lax.fori_loop(0, N // L, zinit, 0)

    def issue_linear(k, b):
      off = base + k * B
      pltpu.async_copy(head_hbm.at[pl.ds(off, B)], headv[b], lsem[b])
      pltpu.async_copy(hq_hbm.at[pl.ds(off, B)], hqv[b], lsem[b])
      pltpu.async_copy(tq_hbm.at[pl.ds(off, B)], tqv[b], lsem[b])

    def wait_linear(k, b):
      off = base + k * B
      pltpu.make_async_copy(head_hbm.at[pl.ds(off, B)], headv[b], lsem[b]).wait()
      pltpu.make_async_copy(hq_hbm.at[pl.ds(off, B)], hqv[b], lsem[b]).wait()
      pltpu.make_async_copy(tq_hbm.at[pl.ds(off, B)], tqv[b], lsem[b]).wait()

    def issue_q(b):
      pltpu.async_copy(q_hbm.at[hqv[b]], qh[b], qsem[b])
      pltpu.async_copy(q_hbm.at[tqv[b]], qt[b], qsem[b])

    def wait_q(b):
      pltpu.make_async_copy(q_hbm.at[hqv[b]], qh[b], qsem[b]).wait()
      pltpu.make_async_copy(q_hbm.at[tqv[b]], qt[b], qsem[b]).wait()

    def wout_desc(k, b):
      off = base + k * B
      return pltpu.make_async_copy(wbuf[b], w_hbm.at[pl.ds(off, B)], wsem[b])

    for kk in range(NB):
      issue_linear(kk, kk)
    wait_linear(0, 0)
    issue_q(0)
    wait_linear(1, 1)
    issue_q(1)

    def do_iter(k, b):
      @pl.when(k + 2 < CH)
      def _():
        wait_linear(k + 2, (b + 2) % NB)
        issue_q((b + 2) % NB)

      wait_q(b)

      @pl.when(k >= NB)
      def _():
        wout_desc(k - NB, b).wait()

      for j in range(B // L):
        s = pl.ds(j * L, L)
        wv = qh[b][s] * qt[b][s]
        wbuf[b][s] = wv
        _scatter_max(smax_loc, hsc, vsc, headv[b][s], wv, lanes)

      pltpu.async_copy(wbuf[b], w_hbm.at[pl.ds(base + k * B, B)], wsem[b])

      @pl.when(k + NB < CH)
      def _():
        issue_linear(k + NB, b)

    def quad(g, c):
      k0 = g * NB
      for u in range(NB):
        do_iter(k0 + u, u)
      return c
    lax.fori_loop(0, CH // NB, quad, 0)
    for u in range(CH % NB):
      do_iter(CH - (CH % NB) + u, (CH - (CH % NB) + u) % NB)

    for i in range(min(NB, CH)):
      k = CH - min(NB, CH) + i
      wout_desc(k, k % NB).wait()
    pltpu.sync_copy(smax_loc, smax_hbm.at[wid])

  mesh = plsc.VectorSubcoreMesh(core_axis_name="c", subcore_axis_name="s",
                                num_cores=NC, num_subcores=NS)
  return pl.kernel(
      body,
      out_type=[jax.ShapeDtypeStruct((E,), jnp.float32),
                jax.ShapeDtypeStruct((NW, N), jnp.float32)],
      mesh=mesh,
      compiler_params=pltpu.CompilerParams(needs_layout_passes=False),
      scratch_types=[
          [pltpu.VMEM((B,), jnp.int32)] * NB,
          [pltpu.VMEM((B,), jnp.int32)] * NB,
          [pltpu.VMEM((B,), jnp.int32)] * NB,
          [pltpu.VMEM((B,), jnp.float32)] * NB,
          [pltpu.VMEM((B,), jnp.float32)] * NB,
          [pltpu.VMEM((B,), jnp.float32)] * NB,
          pltpu.VMEM((N,), jnp.float32),
          pltpu.VMEM((128,), jnp.float32),
          pltpu.VMEM((128,), jnp.float32),
          [pltpu.SemaphoreType.DMA] * NB,
          [pltpu.SemaphoreType.DMA] * NB,
          [pltpu.SemaphoreType.DMA] * NB,
      ],
  )


def _make_mtable(N, R, D):
  BN = 400

  def body(emb_ref, w_ref, out_ref):
    out_ref[...] = emb_ref[...][:, None, :] * w_ref[...][None, :, :]

  return pl.pallas_call(
      body,
      grid=(N // BN,),
      in_specs=[
          pl.BlockSpec((BN, D), lambda n: (n, 0)),
          pl.BlockSpec((R, D), lambda n: (0, 0)),
      ],
      out_specs=pl.BlockSpec((BN, R, D), lambda n: (n, 0, 0)),
      out_shape=jax.ShapeDtypeStruct((N, R, D), jnp.float32),
  )


def _make_phase_c(N, D, R, E):
  EP = E // NW
  CH = EP // B
  STRIPE = (N // NS) // 8 * 8
  TAIL = N - NS * STRIPE

  def body(m_hbm, head_hbm, tq_hbm, we_hbm, smax_hbm, zrows_hbm, zsum_hbm,
           agg_hbm, sump_hbm,
           headv, tqv, wv, shead, pbuf, rows,
           smax_loc, agg_sh, ssum_sh, lsem, gsem, ssem):
    cid = lax.axis_index("c")
    sid = lax.axis_index("s")
    base = (cid * NS + sid) * EP
    lanes = lax.iota(jnp.int32, L)

    pltpu.sync_copy(smax_hbm, smax_loc)

    pltpu.sync_copy(zrows_hbm, agg_sh.at[pl.ds(sid * STRIPE, STRIPE)])

    @pl.when(sid == 0)
    def _():
      pltpu.sync_copy(zrows_hbm.at[pl.ds(0, TAIL)],
                      agg_sh.at[pl.ds(NS * STRIPE, TAIL)])
      pltpu.sync_copy(zsum_hbm, ssum_sh)
    plsc.subcore_barrier()

    def issue_linear(k, b):
      off = base + k * B
      pltpu.async_copy(head_hbm.at[pl.ds(off, B)], headv[b], lsem[b])
      pltpu.async_copy(tq_hbm.at[pl.ds(off, B)], tqv[b], lsem[b])
      pltpu.async_copy(we_hbm.at[pl.ds(off, B)], wv[b], lsem[b])

    def wait_linear(k, b):
      off = base + k * B
      pltpu.make_async_copy(head_hbm.at[pl.ds(off, B)], headv[b], lsem[b]).wait()
      pltpu.make_async_copy(tq_hbm.at[pl.ds(off, B)], tqv[b], lsem[b]).wait()
      pltpu.make_async_copy(we_hbm.at[pl.ds(off, B)], wv[b], lsem[b]).wait()

    def scat_waits(b):
      pltpu.make_async_copy(rows[b], agg_sh.at[shead[b]], ssem[b]).wait()
      pltpu.make_async_copy(pbuf[b], ssum_sh.at[shead[b]], ssem[b]).wait()

    issue_linear(0, 0)
    issue_linear(1, 1)
    wait_linear(0, 0)
    pltpu.async_copy(m_hbm.at[tqv[0]], rows[0], gsem[0])

    def do_iter(k, b):
      nb = 1 - b
      pltpu.make_async_copy(m_hbm.at[tqv[b]], rows[b], gsem[b]).wait()

      @pl.when(k + 1 < CH)
      def _():
        wait_linear(k + 1, nb)

        @pl.when(k >= 1)
        def _():
          scat_waits(nb)
        pltpu.async_copy(m_hbm.at[tqv[nb]], rows[nb], gsem[nb])

      for j in range(B // L):
        s = pl.ds(j * L, L)
        hv = headv[b][s]
        shead[b][s] = hv
        m = plsc.load_gather(smax_loc, [hv])
        pbuf[b][s] = jnp.exp(wv[b][s] - m)

      def edge(i, c2):
        for u in range(2):
          iu = i * 2 + u
          isp = jnp.zeros((L,), jnp.int32) + iu
          psp = plsc.load_gather(pbuf[b], [isp])
          for jj in range(D // L):
            seg = pl.ds(jj * L, L)
            rows[b][iu, seg] = psp * rows[b][iu, seg]
        return c2
      lax.fori_loop(0, B // 2, edge, 0)

      pltpu.async_copy(rows[b], agg_sh.at[shead[b]], ssem[b], add=True)
      pltpu.async_copy(pbuf[b], ssum_sh.at[shead[b]], ssem[b], add=True)

      @pl.when(k + 2 < CH)
      def _():
        issue_linear(k + 2, b)

    def pair(g, c):
      k0 = g * 2
      do_iter(k0, 0)
      do_iter(k0 + 1, 1)
      return c
    lax.fori_loop(0, CH // 2, pair, 0)
    if CH % 2:
      do_iter(CH - 1, 0)

    scat_waits((CH - 2) % 2)
    scat_waits((CH - 1) % 2)

    plsc.subcore_barrier()
    pltpu.sync_copy(agg_sh.at[pl.ds(sid * STRIPE, STRIPE)],
                    agg_hbm.at[pl.ds(cid * N + sid * STRIPE, STRIPE)])

    @pl.when(sid == 0)
    def _():
      pltpu.sync_copy(agg_sh.at[pl.ds(NS * STRIPE, TAIL)],
                      agg_hbm.at[pl.ds(cid * N + NS * STRIPE, TAIL)])
      pltpu.sync_copy(ssum_sh, sump_hbm.at[cid])

  mesh = plsc.VectorSubcoreMesh(core_axis_name="c", subcore_axis_name="s",
                                num_cores=NC, num_subcores=NS)
  return pl.kernel(
      body,
      out_type=[jax.ShapeDtypeStruct((NC * N, D), jnp.float32),
                jax.ShapeDtypeStruct((NC, N), jnp.float32)],
      mesh=mesh,
      compiler_params=pltpu.CompilerParams(needs_layout_passes=False),
      scratch_types=[
          [pltpu.VMEM((B,), jnp.int32)] * 2,
          [pltpu.VMEM((B,), jnp.int32)] * 2,
          [pltpu.VMEM((B,), jnp.float32)] * 2,
          [pltpu.VMEM((B,), jnp.int32)] * 2,
          [pltpu.VMEM((B,), jnp.float32)] * 2,
          [pltpu.VMEM((B, D), jnp.float32)] * 2,
          pltpu.VMEM((N,), jnp.float32),
          pltpu.VMEM_SHARED((N, D), jnp.float32),
          pltpu.VMEM_SHARED((N,), jnp.float32),
          [pltpu.SemaphoreType.DMA] * 2,
          [pltpu.SemaphoreType.DMA] * 2,
          [pltpu.SemaphoreType.DMA] * 2,
      ],
  )


def _make_matmuls(U, N, D):
  BU = 256
  UB = U // BU

  def body(im_ref, emb_ref, uemb_ref, uout_ref, dout_ref):
    u = pl.program_id(0)
    im = im_ref[...]
    uout_ref[...] = lax.dot_general(im, emb_ref[...], (((1,), (0,)), ((), ())),
                                    preferred_element_type=jnp.float32)
    prod_d = lax.dot_general(im, uemb_ref[...], (((0,), (0,)), ((), ())),
                             preferred_element_type=jnp.float32)

    @pl.when(u == 0)
    def _():
      dout_ref[...] = prod_d

    @pl.when(u != 0)
    def _():
      dout_ref[...] += prod_d

  return pl.pallas_call(
      body,
      grid=(UB,),
      in_specs=[
          pl.BlockSpec((BU, N), lambda u: (u, 0)),
          pl.BlockSpec((N, D), lambda u: (0, 0)),
          pl.BlockSpec((BU, D), lambda u: (u, 0)),
      ],
      out_specs=[
          pl.BlockSpec((BU, D), lambda u: (u, 0)),
          pl.BlockSpec((N, D), lambda u: (0, 0)),
      ],
      out_shape=[jax.ShapeDtypeStruct((U, D), jnp.float32),
                 jax.ShapeDtypeStruct((N, D), jnp.float32)],
  )


def _make_combine(N, D):
  def body(dense_ref, agg_ref, sum_ref, out_ref):
    s = sum_ref[0] + sum_ref[1]
    a = agg_ref[0] + agg_ref[1]
    out_ref[...] = dense_ref[...] + a * (1.0 / (s + EPS))[:, None]

  return pl.pallas_call(
      body,
      out_shape=jax.ShapeDtypeStruct((N, D), jnp.float32),
  )


@jax.jit
def kernel(entity_emb, user_emb, edge_index, edge_type, interact_mat, weight):
  N, D = entity_emb.shape
  U = user_emb.shape[0]
  E = edge_index.shape[1]
  R = weight.shape[0]

  head = edge_index[0]
  tail = edge_index[1]
  rel = edge_type - 1

  q = _q_table(entity_emb, weight)
  hq_idx = head * R + rel
  tq_idx = tail * R + rel
  w_e, smax_parts = _make_phase_a(N, R, E)(q.reshape(-1), head, hq_idx, tq_idx)
  seg_max = _reduce_max(smax_parts)
  zrows = jnp.zeros(((N // NS) // 8 * 8, D), jnp.float32)
  zsum = jnp.zeros((N,), jnp.float32)
  m_table = _make_mtable(N, R, D)(entity_emb, weight).reshape(N * R, D)
  agg_flat, sum_parts = _make_phase_c(N, D, R, E)(
      m_table, head, tq_idx, w_e, seg_max, zrows, zsum)
  agg_parts = agg_flat.reshape(NC, N, D)
  user_agg, dense = _make_matmuls(U, N, D)(interact_mat, entity_emb, user_emb)
  entity_agg = _make_combine(N, D)(dense, agg_parts, sum_parts)
  return entity_agg, user_agg


# speedup vs baseline: 17.5506x; 1.0075x over previous
import functools
import jax
import jax.numpy as jnp
from jax import lax
from jax.experimental import pallas as pl
from jax.experimental.pallas import tpu as pltpu
from jax.experimental.pallas import tpu_sc as plsc

NC, NS, L = 2, 16, 16
NW = NC * NS
B = 80
EPS = 1e-16
HIGH = lax.Precision.HIGHEST


def _dedup_combine(hsc, vsc, hv, vv, lanes, is_add):
  hvf = hv.astype(jnp.float32)
  hsc[pl.ds(0, L)] = hvf
  vsc[pl.ds(0, L)] = vv
  acc = vv
  first = jnp.ones((L,), jnp.bool_)
  for s in range(1, L):
    idx = (lanes + s) & (L - 1)
    hr = plsc.load_gather(hsc, [idx])
    vr = plsc.load_gather(vsc, [idx])
    same = hr == hvf
    comb = acc + vr if is_add else jnp.maximum(acc, vr)
    acc = jnp.where(same, comb, acc)
    first = first & jnp.logical_not(same & (idx < lanes))
  return acc, first


def _scatter_max(tab, hsc, vsc, hv, vv, lanes):
  acc, first = _dedup_combine(hsc, vsc, hv, vv, lanes, False)
  cur = plsc.load_gather(tab, [hv])
  plsc.store_scatter(tab, [hv], jnp.maximum(acc, cur), mask=first)


def _scatter_add(tab, hsc, vsc, hv, vv, lanes):
  acc, first = _dedup_combine(hsc, vsc, hv, vv, lanes, True)
  cur = plsc.load_gather(tab, [hv])
  plsc.store_scatter(tab, [hv], cur + acc, mask=first)


def _q_body(emb_ref, w_ref, q_ref):
  e2 = emb_ref[...] * emb_ref[...]
  w2 = w_ref[...] * w_ref[...]
  q_ref[...] = lax.dot_general(e2, w2, (((1,), (1,)), ((), ())),
                               precision=HIGH,
                               preferred_element_type=jnp.float32)


def _q_table(entity_emb, weight):
  N, _ = entity_emb.shape
  R = weight.shape[0]
  return pl.pallas_call(
      _q_body,
      out_shape=jax.ShapeDtypeStruct((N, R), jnp.float32),
  )(entity_emb, weight)


def _maxred_body(parts_ref, out_ref):
  out_ref[...] = jnp.max(parts_ref[...], axis=0)


def _reduce_max(parts):
  _, N = parts.shape
  return pl.pallas_call(
      _maxred_body,
      out_shape=jax.ShapeDtypeStruct((N,), jnp.float32),
  )(parts)


def _make_phase_a(N, R, E):
  EP = E // NW
  CH = EP // B
  NB = 4

  def body(q_hbm, head_hbm, hq_hbm, tq_hbm, w_hbm, smax_hbm,
           headv, hqv, tqv, qh, qt, wbuf, smax_loc, hsc, vsc,
           lsem, qsem, wsem):
    cid = lax.axis_index("c")
    sid = lax.axis_index("s")
    wid = cid * NS + sid
    base = wid * EP
    lanes = lax.iota(jnp.int32, L)
    zeros = jnp.zeros((L,), jnp.float32)

    def zinit(i, c):
      smax_loc[pl.ds(i * L, L)] = zeros
      return c
    lax.fori_loop(0, N // L, zinit, 0)

    def issue_linear(k, b):
      off = base + k * B
      pltpu.async_copy(head_hbm.at[pl.ds(off, B)], headv[b], lsem[b])
      pltpu.async_copy(hq_hbm.at[pl.ds(off, B)], hqv[b], lsem[b])
      pltpu.async_copy(tq_hbm.at[pl.ds(off, B)], tqv[b], lsem[b])

    def wait_linear(k, b):
      off = base + k * B
      pltpu.make_async_copy(head_hbm.at[pl.ds(off, B)], headv[b], lsem[b]).wait()
      pltpu.make_async_copy(hq_hbm.at[pl.ds(off, B)], hqv[b], lsem[b]).wait()
      pltpu.make_async_copy(tq_hbm.at[pl.ds(off, B)], tqv[b], lsem[b]).wait()

    def issue_q(b):
      pltpu.async_copy(q_hbm.at[hqv[b]], qh[b], qsem[b])
      pltpu.async_copy(q_hbm.at[tqv[b]], qt[b], qsem[b])

    def wait_q(b):
      pltpu.make_async_copy(q_hbm.at[hqv[b]], qh[b], qsem[b]).wait()
      pltpu.make_async_copy(q_hbm.at[tqv[b]], qt[b], qsem[b]).wait()

    def wout_desc(k, b):
      off = base + k * B
      return pltpu.make_async_copy(wbuf[b], w_hbm.at[pl.ds(off, B)], wsem[b])

    for kk in range(NB):
      issue_linear(kk, kk)
    wait_linear(0, 0)
    issue_q(0)
    wait_linear(1, 1)
    issue_q(1)

    def do_iter(k, b):
      @pl.when(k + 2 < CH)
      def _():
        wait_linear(k + 2, (b + 2) % NB)
        issue_q((b + 2) % NB)

      wait_q(b)

      @pl.when(k >= NB)
      def _():
        wout_desc(k - NB, b).wait()

      for j in range(B // L):
        s = pl.ds(j * L, L)
        wv = qh[b][s] * qt[b][s]
        wbuf[b][s] = wv
        _scatter_max(smax_loc, hsc, vsc, headv[b][s], wv, lanes)

      pltpu.async_copy(wbuf[b], w_hbm.at[pl.ds(base + k * B, B)], wsem[b])

      @pl.when(k + NB < CH)
      def _():
        issue_linear(k + NB, b)

    def quad(g, c):
      k0 = g * NB
      for u in range(NB):
        do_iter(k0 + u, u)
      return c
    lax.fori_loop(0, CH // NB, quad, 0)
    for u in range(CH % NB):
      do_iter(CH - (CH % NB) + u, (CH - (CH % NB) + u) % NB)

    for i in range(min(NB, CH)):
      k = CH - min(NB, CH) + i
      wout_desc(k, k % NB).wait()
    pltpu.sync_copy(smax_loc, smax_hbm.at[wid])

  mesh = plsc.VectorSubcoreMesh(core_axis_name="c", subcore_axis_name="s",
                                num_cores=NC, num_subcores=NS)
  return pl.kernel(
      body,
      out_type=[jax.ShapeDtypeStruct((E,), jnp.float32),
                jax.ShapeDtypeStruct((NW, N), jnp.float32)],
      mesh=mesh,
      compiler_params=pltpu.CompilerParams(needs_layout_passes=False),
      scratch_types=[
          [pltpu.VMEM((B,), jnp.int32)] * NB,
          [pltpu.VMEM((B,), jnp.int32)] * NB,
          [pltpu.VMEM((B,), jnp.int32)] * NB,
          [pltpu.VMEM((B,), jnp.float32)] * NB,
          [pltpu.VMEM((B,), jnp.float32)] * NB,
          [pltpu.VMEM((B,), jnp.float32)] * NB,
          pltpu.VMEM((N,), jnp.float32),
          pltpu.VMEM((128,), jnp.float32),
          pltpu.VMEM((128,), jnp.float32),
          [pltpu.SemaphoreType.DMA] * NB,
          [pltpu.SemaphoreType.DMA] * NB,
          [pltpu.SemaphoreType.DMA] * NB,
      ],
  )


def _make_mtable(N, R, D):
  BN = 400

  def body(emb_ref, w_ref, out_ref):
    out_ref[...] = emb_ref[...][:, None, :] * w_ref[...][None, :, :]

  return pl.pallas_call(
      body,
      grid=(N // BN,),
      in_specs=[
          pl.BlockSpec((BN, D), lambda n: (n, 0)),
          pl.BlockSpec((R, D), lambda n: (0, 0)),
      ],
      out_specs=pl.BlockSpec((BN, R, D), lambda n: (n, 0, 0)),
      out_shape=jax.ShapeDtypeStruct((N, R, D), jnp.float32),
  )


def _make_phase_c(N, D, R, E):
  EP = E // NW
  CH = EP // B
  STRIPE = (N // NS) // 8 * 8
  TAIL = N - NS * STRIPE

  def body(m_hbm, head_hbm, tq_hbm, we_hbm, smax_hbm, zrows_hbm, zsum_hbm,
           agg_hbm, sump_hbm,
           headv, tqv, wv, shead, pbuf, rows,
           smax_loc, agg_sh, ssum_sh, lsem, gsem, ssem):
    cid = lax.axis_index("c")
    sid = lax.axis_index("s")
    base = (cid * NS + sid) * EP
    lanes = lax.iota(jnp.int32, L)

    pltpu.sync_copy(smax_hbm, smax_loc)

    pltpu.sync_copy(zrows_hbm, agg_sh.at[pl.ds(sid * STRIPE, STRIPE)])

    @pl.when(sid == 0)
    def _():
      pltpu.sync_copy(zrows_hbm.at[pl.ds(0, TAIL)],
                      agg_sh.at[pl.ds(NS * STRIPE, TAIL)])
      pltpu.sync_copy(zsum_hbm, ssum_sh)
    plsc.subcore_barrier()

    def issue_linear(k, b):
      off = base + k * B
      pltpu.async_copy(head_hbm.at[pl.ds(off, B)], headv[b], lsem[b])
      pltpu.async_copy(tq_hbm.at[pl.ds(off, B)], tqv[b], lsem[b])
      pltpu.async_copy(we_hbm.at[pl.ds(off, B)], wv[b], lsem[b])

    def wait_linear(k, b):
      off = base + k * B
      pltpu.make_async_copy(head_hbm.at[pl.ds(off, B)], headv[b], lsem[b]).wait()
      pltpu.make_async_copy(tq_hbm.at[pl.ds(off, B)], tqv[b], lsem[b]).wait()
      pltpu.make_async_copy(we_hbm.at[pl.ds(off, B)], wv[b], lsem[b]).wait()

    def scat_waits(b):
      pltpu.make_async_copy(rows[b], agg_sh.at[shead[b]], ssem[b]).wait()
      pltpu.make_async_copy(pbuf[b], ssum_sh.at[shead[b]], ssem[b]).wait()

    issue_linear(0, 0)
    issue_linear(1, 1)
    issue_linear(2, 2)
    wait_linear(0, 0)
    pltpu.async_copy(m_hbm.at[tqv[0]], rows[0], gsem[0])
    wait_linear(1, 1)
    pltpu.async_copy(m_hbm.at[tqv[1]], rows[1], gsem[1])

    def do_iter(k, b):
      bp2 = (b + 2) % 3

      @pl.when(k + 2 < CH)
      def _():
        wait_linear(k + 2, bp2)

        @pl.when(k >= 1)
        def _():
          scat_waits(bp2)
        pltpu.async_copy(m_hbm.at[tqv[bp2]], rows[bp2], gsem[bp2])

      pltpu.make_async_copy(m_hbm.at[tqv[b]], rows[b], gsem[b]).wait()

      for j in range(B // L):
        s = pl.ds(j * L, L)
        hv = headv[b][s]
        shead[b][s] = hv
        m = plsc.load_gather(smax_loc, [hv])
        pbuf[b][s] = jnp.exp(wv[b][s] - m)

      def edge(i, c2):
        for u in range(2):
          iu = i * 2 + u
          isp = jnp.zeros((L,), jnp.int32) + iu
          psp = plsc.load_gather(pbuf[b], [isp])
          for jj in range(D // L):
            seg = pl.ds(jj * L, L)
            rows[b][iu, seg] = psp * rows[b][iu, seg]
        return c2
      lax.fori_loop(0, B // 2, edge, 0)

      pltpu.async_copy(rows[b], agg_sh.at[shead[b]], ssem[b], add=True)
      pltpu.async_copy(pbuf[b], ssum_sh.at[shead[b]], ssem[b], add=True)

      @pl.when(k + 3 < CH)
      def _():
        issue_linear(k + 3, b)

    def trip(g, c):
      k0 = g * 3
      do_iter(k0, 0)
      do_iter(k0 + 1, 1)
      do_iter(k0 + 2, 2)
      return c
    lax.fori_loop(0, CH // 3, trip, 0)
    for u in range(CH % 3):
      do_iter(CH - (CH % 3) + u, (CH - (CH % 3) + u) % 3)

    scat_waits((CH - 2) % 3)
    scat_waits((CH - 1) % 3)

    plsc.subcore_barrier()
    pltpu.sync_copy(agg_sh.at[pl.ds(sid * STRIPE, STRIPE)],
                    agg_hbm.at[pl.ds(cid * N + sid * STRIPE, STRIPE)])

    @pl.when(sid == 0)
    def _():
      pltpu.sync_copy(agg_sh.at[pl.ds(NS * STRIPE, TAIL)],
                      agg_hbm.at[pl.ds(cid * N + NS * STRIPE, TAIL)])
      pltpu.sync_copy(ssum_sh, sump_hbm.at[cid])

  mesh = plsc.VectorSubcoreMesh(core_axis_name="c", subcore_axis_name="s",
                                num_cores=NC, num_subcores=NS)
  return pl.kernel(
      body,
      out_type=[jax.ShapeDtypeStruct((NC * N, D), jnp.float32),
                jax.ShapeDtypeStruct((NC, N), jnp.float32)],
      mesh=mesh,
      compiler_params=pltpu.CompilerParams(needs_layout_passes=False),
      scratch_types=[
          [pltpu.VMEM((B,), jnp.int32)] * 3,
          [pltpu.VMEM((B,), jnp.int32)] * 3,
          [pltpu.VMEM((B,), jnp.float32)] * 3,
          [pltpu.VMEM((B,), jnp.int32)] * 3,
          [pltpu.VMEM((B,), jnp.float32)] * 3,
          [pltpu.VMEM((B, D), jnp.float32)] * 3,
          pltpu.VMEM((N,), jnp.float32),
          pltpu.VMEM_SHARED((N, D), jnp.float32),
          pltpu.VMEM_SHARED((N,), jnp.float32),
          [pltpu.SemaphoreType.DMA] * 3,
          [pltpu.SemaphoreType.DMA] * 3,
          [pltpu.SemaphoreType.DMA] * 3,
      ],
  )


def _make_matmuls(U, N, D):
  BU = 256
  UB = U // BU

  def body(im_ref, emb_ref, uemb_ref, uout_ref, dout_ref):
    u = pl.program_id(0)
    im = im_ref[...]
    uout_ref[...] = lax.dot_general(im, emb_ref[...], (((1,), (0,)), ((), ())),
                                    preferred_element_type=jnp.float32)
    prod_d = lax.dot_general(im, uemb_ref[...], (((0,), (0,)), ((), ())),
                             preferred_element_type=jnp.float32)

    @pl.when(u == 0)
    def _():
      dout_ref[...] = prod_d

    @pl.when(u != 0)
    def _():
      dout_ref[...] += prod_d

  return pl.pallas_call(
      body,
      grid=(UB,),
      in_specs=[
          pl.BlockSpec((BU, N), lambda u: (u, 0)),
          pl.BlockSpec((N, D), lambda u: (0, 0)),
          pl.BlockSpec((BU, D), lambda u: (u, 0)),
      ],
      out_specs=[
          pl.BlockSpec((BU, D), lambda u: (u, 0)),
          pl.BlockSpec((N, D), lambda u: (0, 0)),
      ],
      out_shape=[jax.ShapeDtypeStruct((U, D), jnp.float32),
                 jax.ShapeDtypeStruct((N, D), jnp.float32)],
  )


def _make_combine(N, D):
  def body(dense_ref, agg_ref, sum_ref, out_ref):
    s = sum_ref[0] + sum_ref[1]
    a = agg_ref[0] + agg_ref[1]
    out_ref[...] = dense_ref[...] + a * (1.0 / (s + EPS))[:, None]

  return pl.pallas_call(
      body,
      out_shape=jax.ShapeDtypeStruct((N, D), jnp.float32),
  )


@jax.jit
def kernel(entity_emb, user_emb, edge_index, edge_type, interact_mat, weight):
  N, D = entity_emb.shape
  U = user_emb.shape[0]
  E = edge_index.shape[1]
  R = weight.shape[0]

  head = edge_index[0]
  tail = edge_index[1]
  rel = edge_type - 1

  q = _q_table(entity_emb, weight)
  hq_idx = head * R + rel
  tq_idx = tail * R + rel
  w_e, smax_parts = _make_phase_a(N, R, E)(q.reshape(-1), head, hq_idx, tq_idx)
  seg_max = _reduce_max(smax_parts)
  zrows = jnp.zeros(((N // NS) // 8 * 8, D), jnp.float32)
  zsum = jnp.zeros((N,), jnp.float32)
  m_table = _make_mtable(N, R, D)(entity_emb, weight).reshape(N * R, D)
  agg_flat, sum_parts = _make_phase_c(N, D, R, E)(
      m_table, head, tq_idx, w_e, seg_max, zrows, zsum)
  agg_parts = agg_flat.reshape(NC, N, D)
  user_agg, dense = _make_matmuls(U, N, D)(interact_mat, entity_emb, user_emb)
  entity_agg = _make_combine(N, D)(dense, agg_parts, sum_parts)
  return entity_agg, user_agg
